# Initial kernel scaffold; baseline (speedup 1.0000x reference)
import functools

import jax
import jax.numpy as jnp
from jax import lax
from jax.experimental import pallas as pl
from jax.experimental.pallas import tpu as pltpu
from jax.experimental.pallas import tpu_sc as plsc

N = 10000
D = 128
H = 128
R = 3
E = 160000

NC = 2
NS = 16
NW = NC * NS
EPT = E // NW
CH = 128
NCH = -(-EPT // CH)
EPT_PAD = NCH * CH
NPAD = N + 16
ZROWS = NPAD // NS

_BN_SCALE = float(1.0 / (1.0 + 1e-5) ** 0.5)

BLK = 1000
GRID = N // BLK


def _make_sc_agg(with_counts):
  mesh = plsc.VectorSubcoreMesh(core_axis_name="c", subcore_axis_name="s")
  out_type = [jax.ShapeDtypeStruct((R, NC, NPAD, D), jnp.float32)]
  scratch = [
      pltpu.VMEM((NCH, CH), jnp.int32),
      pltpu.VMEM((NCH, CH), jnp.int32),
      pltpu.VMEM((CH, D), jnp.float32),
      pltpu.VMEM_SHARED((NPAD, D), jnp.float32),
      pltpu.SemaphoreType.DMA,
  ]
  if with_counts:
    out_type.append(jax.ShapeDtypeStruct((R, NC, NPAD, 16), jnp.float32))
    scratch += [
        pltpu.VMEM((CH, 16), jnp.float32),
        pltpu.VMEM_SHARED((NPAD, 16), jnp.float32),
    ]

  @functools.partial(pl.kernel, mesh=mesh, out_type=tuple(out_type),
                     scratch_types=tuple(scratch))
  def sc_agg(*refs):
    if with_counts:
      (feat, src, dst, zfeat, zcnt, ones, sums_out, cnt_out,
       src_v, dst_v, rows_v, acc_s, sem, ones_v, cnt_s) = refs
    else:
      (feat, src, dst, zfeat, sums_out,
       src_v, dst_v, rows_v, acc_s, sem) = refs
    c = lax.axis_index("c")
    s = lax.axis_index("s")
    wid = c * NS + s
    my_rows = pl.ds(s * ZROWS, ZROWS)
    if with_counts:
      pltpu.sync_copy(ones, ones_v)
    for r in range(R):
      pltpu.sync_copy(zfeat, acc_s.at[my_rows])
      if with_counts:
        pltpu.sync_copy(zcnt, cnt_s.at[my_rows])
      pltpu.sync_copy(src.at[r, wid], src_v)
      pltpu.sync_copy(dst.at[r, wid], dst_v)
      plsc.subcore_barrier()

      def chunk(j, carry):
        pltpu.async_copy(feat.at[src_v.at[j]], rows_v, sem).wait()
        pltpu.sync_copy(rows_v, acc_s.at[dst_v.at[j]], add=True)
        if with_counts:
          pltpu.sync_copy(ones_v, cnt_s.at[dst_v.at[j]], add=True)
        return carry

      lax.fori_loop(0, NCH, chunk, 0)
      plsc.subcore_barrier()
      pltpu.sync_copy(acc_s.at[my_rows], sums_out.at[r, c, my_rows])
      if with_counts:
        pltpu.sync_copy(cnt_s.at[my_rows], cnt_out.at[r, c, my_rows])
      plsc.subcore_barrier()

  return sc_agg


_sc_agg_counts = _make_sc_agg(True)
_sc_agg_plain = _make_sc_agg(False)


def _leaky(v):
  return jnp.where(v >= 0, v, 0.01 * v)


def _hetero_acc(feat, sums_ref, cnt_ref, ws_ref, wn_ref, b_ref):
  acc = jnp.zeros(feat.shape, jnp.float32)
  for r in range(R):
    acc += jnp.dot(feat, ws_ref[r], preferred_element_type=jnp.float32)
    sm = sums_ref[r, 0] + sums_ref[r, 1]
    ct = cnt_ref[r, 0][:, :1] + cnt_ref[r, 1][:, :1]
    hn = sm * (1.0 / jnp.maximum(ct, 1.0))
    acc += jnp.dot(hn, wn_ref[r], preferred_element_type=jnp.float32)
    acc += b_ref[r]
  return acc


def _layer1_body(x_ref, sums_ref, cnt_ref, ws_ref, wn_ref, b_ref, g_ref,
                 bt_ref, out_ref):
  acc = _hetero_acc(x_ref[...], sums_ref, cnt_ref, ws_ref, wn_ref, b_ref)
  out_ref[...] = _leaky(g_ref[...] * _BN_SCALE * acc + bt_ref[...])


def _layer2_body(h1_ref, sums_ref, cnt_ref, ws_ref, wn_ref, b_ref, g_ref,
                 bt_ref, fw_ref, fb_ref, out_ref):
  h1 = h1_ref[...]
  acc = _hetero_acc(h1, sums_ref, cnt_ref, ws_ref, wn_ref, b_ref)
  h2 = _leaky(g_ref[...] * _BN_SCALE * acc + bt_ref[...] + h1)
  out_ref[...] = (jnp.dot(h1, fw_ref[0], preferred_element_type=jnp.float32)
                  + jnp.dot(h2, fw_ref[1], preferred_element_type=jnp.float32)
                  + fb_ref[...])


_COMMON_SPECS = [
    pl.BlockSpec((BLK, D), lambda i: (i, 0)),
    pl.BlockSpec((R, NC, BLK, D), lambda i: (0, 0, i, 0)),
    pl.BlockSpec((R, NC, BLK, 16), lambda i: (0, 0, i, 0)),
    pl.BlockSpec((R, D, H), lambda i: (0, 0, 0)),
    pl.BlockSpec((R, D, H), lambda i: (0, 0, 0)),
    pl.BlockSpec((R, 1, H), lambda i: (0, 0, 0)),
    pl.BlockSpec((1, H), lambda i: (0, 0)),
    pl.BlockSpec((1, H), lambda i: (0, 0)),
]

_layer1_tc = pl.pallas_call(
    _layer1_body,
    grid=(GRID,),
    in_specs=_COMMON_SPECS,
    out_specs=pl.BlockSpec((BLK, H), lambda i: (i, 0)),
    out_shape=jax.ShapeDtypeStruct((N, H), jnp.float32),
)

_layer2_tc = pl.pallas_call(
    _layer2_body,
    grid=(GRID,),
    in_specs=_COMMON_SPECS + [
        pl.BlockSpec((2, H, H), lambda i: (0, 0, 0)),
        pl.BlockSpec((1, H), lambda i: (0, 0)),
    ],
    out_specs=pl.BlockSpec((BLK, H), lambda i: (i, 0)),
    out_shape=jax.ShapeDtypeStruct((N, H), jnp.float32),
)


def _prep_edges(ei):
  pad = EPT_PAD - EPT
  src = jnp.pad(ei[0].reshape(NW, EPT), ((0, 0), (0, pad)))
  dst = jnp.pad(ei[1].reshape(NW, EPT), ((0, 0), (0, pad)),
                constant_values=N)
  return src.reshape(NW, NCH, CH), dst.reshape(NW, NCH, CH)


def kernel(x, edge_index_0, edge_index_1, edge_index_2, W_self1, W_neigh1,
           b1, W_self2, W_neigh2, b2, bn1_gamma, bn1_beta, bn2_gamma,
           bn2_beta, fuse_W, fuse_b):
  pairs = [_prep_edges(e) for e in (edge_index_0, edge_index_1, edge_index_2)]
  src = jnp.stack([p[0] for p in pairs])
  dst = jnp.stack([p[1] for p in pairs])
  zfeat = jnp.zeros((ZROWS, D), jnp.float32)
  zcnt = jnp.zeros((ZROWS, 16), jnp.float32)
  ones = jnp.ones((CH, 16), jnp.float32)

  sums1, cnt = _sc_agg_counts(x, src, dst, zfeat, zcnt, ones)
  h1 = _layer1_tc(x, sums1, cnt, W_self1, W_neigh1, b1.reshape(R, 1, H),
                  bn1_gamma.reshape(1, H), bn1_beta.reshape(1, H))
  sums2 = _sc_agg_plain(h1, src, dst, zfeat)
  out = _layer2_tc(h1, sums2, cnt, W_self2, W_neigh2, b2.reshape(R, 1, H),
                   bn2_gamma.reshape(1, H), bn2_beta.reshape(1, H),
                   fuse_W.reshape(2, H, H), fuse_b.reshape(1, H))
  return out


# speedup vs baseline: 2.6352x; 2.6352x over previous
import functools

import jax
import jax.numpy as jnp
from jax import lax
from jax.experimental import pallas as pl
from jax.experimental.pallas import tpu as pltpu
from jax.experimental.pallas import tpu_sc as plsc

N = 10000
D = 128
H = 128
R = 3
E = 160000

NC = 2
NS = 16
NW = NC * NS
EPT = E // NW
CH = 128
NCH = -(-EPT // CH)
EPT_PAD = NCH * CH
NPAD = 10112
ZROWS = NPAD // NS

_BN_SCALE = float(1.0 / (1.0 + 1e-5) ** 0.5)

BLK = 1000
GRID = N // BLK


def _make_sc_agg(with_counts):
  mesh = plsc.VectorSubcoreMesh(core_axis_name="c", subcore_axis_name="s")
  out_type = [jax.ShapeDtypeStruct((R, NC, NPAD, D), jnp.float32)]
  scratch = [
      pltpu.VMEM((NCH, CH), jnp.int32),
      pltpu.VMEM((NCH, CH), jnp.int32),
      pltpu.VMEM((CH, D), jnp.float32),
      pltpu.VMEM_SHARED((NPAD, D), jnp.float32),
      pltpu.SemaphoreType.DMA,
  ]
  if with_counts:
    out_type.append(jax.ShapeDtypeStruct((R, NC, NPAD, D), jnp.float32))
    scratch += [pltpu.VMEM((CH, D), jnp.float32)]

  @functools.partial(pl.kernel, mesh=mesh, out_type=tuple(out_type),
                     scratch_types=tuple(scratch))
  def sc_agg(*refs):
    if with_counts:
      (feat, src, dst, zfeat, ones, sums_out, cnt_out,
       src_v, dst_v, rows_v, acc_s, sem, ones_v) = refs
    else:
      (feat, src, dst, zfeat, sums_out,
       src_v, dst_v, rows_v, acc_s, sem) = refs
    c = lax.axis_index("c")
    s = lax.axis_index("s")
    wid = c * NS + s
    my_rows = pl.ds(s * ZROWS, ZROWS)
    if with_counts:
      pltpu.sync_copy(ones, ones_v)
    for r in range(R):
      pltpu.sync_copy(zfeat, acc_s.at[my_rows])
      pltpu.sync_copy(src.at[r, wid], src_v)
      pltpu.sync_copy(dst.at[r, wid], dst_v)
      plsc.subcore_barrier()

      def chunk(j, carry):
        pltpu.async_copy(feat.at[src_v.at[j]], rows_v, sem).wait()
        pltpu.sync_copy(rows_v, acc_s.at[dst_v.at[j]], add=True)
        return carry

      lax.fori_loop(0, NCH, chunk, 0)
      plsc.subcore_barrier()
      pltpu.sync_copy(acc_s.at[my_rows], sums_out.at[r, c, my_rows])
      plsc.subcore_barrier()
    if with_counts:
      for r in range(R):
        pltpu.sync_copy(zfeat, acc_s.at[my_rows])
        pltpu.sync_copy(dst.at[r, wid], dst_v)
        plsc.subcore_barrier()

        def cchunk(j, carry):
          pltpu.sync_copy(ones_v, acc_s.at[dst_v.at[j]], add=True)
          return carry

        lax.fori_loop(0, NCH, cchunk, 0)
        plsc.subcore_barrier()
        pltpu.sync_copy(acc_s.at[my_rows], cnt_out.at[r, c, my_rows])
        plsc.subcore_barrier()

  return sc_agg


_sc_agg_counts = _make_sc_agg(True)
_sc_agg_plain = _make_sc_agg(False)


def _leaky(v):
  return jnp.where(v >= 0, v, 0.01 * v)


def _hetero_acc(feat, sums_ref, cnt_ref, ws_ref, wn_ref, b_ref):
  acc = jnp.zeros(feat.shape, jnp.float32)
  for r in range(R):
    acc += jnp.dot(feat, ws_ref[r], preferred_element_type=jnp.float32)
    sm = sums_ref[r, 0] + sums_ref[r, 1]
    ct = cnt_ref[r, 0][:, :1] + cnt_ref[r, 1][:, :1]
    hn = sm * (1.0 / jnp.maximum(ct, 1.0))
    acc += jnp.dot(hn, wn_ref[r], preferred_element_type=jnp.float32)
    acc += b_ref[r]
  return acc


def _layer1_body(x_ref, sums_ref, cnt_ref, ws_ref, wn_ref, b_ref, g_ref,
                 bt_ref, out_ref):
  acc = _hetero_acc(x_ref[...], sums_ref, cnt_ref, ws_ref, wn_ref, b_ref)
  out_ref[...] = _leaky(g_ref[...] * _BN_SCALE * acc + bt_ref[...])


def _layer2_body(h1_ref, sums_ref, cnt_ref, ws_ref, wn_ref, b_ref, g_ref,
                 bt_ref, fw_ref, fb_ref, out_ref):
  h1 = h1_ref[...]
  acc = _hetero_acc(h1, sums_ref, cnt_ref, ws_ref, wn_ref, b_ref)
  h2 = _leaky(g_ref[...] * _BN_SCALE * acc + bt_ref[...] + h1)
  out_ref[...] = (jnp.dot(h1, fw_ref[0], preferred_element_type=jnp.float32)
                  + jnp.dot(h2, fw_ref[1], preferred_element_type=jnp.float32)
                  + fb_ref[...])


_COMMON_SPECS = [
    pl.BlockSpec((BLK, D), lambda i: (i, 0)),
    pl.BlockSpec((R, NC, BLK, D), lambda i: (0, 0, i, 0)),
    pl.BlockSpec((R, NC, BLK, D), lambda i: (0, 0, i, 0)),
    pl.BlockSpec((R, D, H), lambda i: (0, 0, 0)),
    pl.BlockSpec((R, D, H), lambda i: (0, 0, 0)),
    pl.BlockSpec((R, 1, H), lambda i: (0, 0, 0)),
    pl.BlockSpec((1, H), lambda i: (0, 0)),
    pl.BlockSpec((1, H), lambda i: (0, 0)),
]

_layer1_tc = pl.pallas_call(
    _layer1_body,
    grid=(GRID,),
    in_specs=_COMMON_SPECS,
    out_specs=pl.BlockSpec((BLK, H), lambda i: (i, 0)),
    out_shape=jax.ShapeDtypeStruct((N, H), jnp.float32),
)

_layer2_tc = pl.pallas_call(
    _layer2_body,
    grid=(GRID,),
    in_specs=_COMMON_SPECS + [
        pl.BlockSpec((2, H, H), lambda i: (0, 0, 0)),
        pl.BlockSpec((1, H), lambda i: (0, 0)),
    ],
    out_specs=pl.BlockSpec((BLK, H), lambda i: (i, 0)),
    out_shape=jax.ShapeDtypeStruct((N, H), jnp.float32),
)


def _prep_edges(ei):
  pad = EPT_PAD - EPT
  src = jnp.pad(ei[0].reshape(NW, EPT), ((0, 0), (0, pad)))
  dst = jnp.pad(ei[1].reshape(NW, EPT), ((0, 0), (0, pad)),
                constant_values=N)
  return src.reshape(NW, NCH, CH), dst.reshape(NW, NCH, CH)


def kernel(x, edge_index_0, edge_index_1, edge_index_2, W_self1, W_neigh1,
           b1, W_self2, W_neigh2, b2, bn1_gamma, bn1_beta, bn2_gamma,
           bn2_beta, fuse_W, fuse_b):
  pairs = [_prep_edges(e) for e in (edge_index_0, edge_index_1, edge_index_2)]
  src = jnp.stack([p[0] for p in pairs])
  dst = jnp.stack([p[1] for p in pairs])
  zfeat = jnp.zeros((ZROWS, D), jnp.float32)
  ones = jnp.ones((CH, D), jnp.float32)

  sums1, cnt = _sc_agg_counts(x, src, dst, zfeat, ones)
  h1 = _layer1_tc(x, sums1, cnt, W_self1, W_neigh1, b1.reshape(R, 1, H),
                  bn1_gamma.reshape(1, H), bn1_beta.reshape(1, H))
  (sums2,) = _sc_agg_plain(h1, src, dst, zfeat)
  out = _layer2_tc(h1, sums2, cnt, W_self2, W_neigh2, b2.reshape(R, 1, H),
                   bn2_gamma.reshape(1, H), bn2_beta.reshape(1, H),
                   fuse_W.reshape(2, H, H), fuse_b.reshape(1, H))
  return out


# speedup vs baseline: 2.8033x; 1.0638x over previous
import functools

import jax
import jax.numpy as jnp
from jax import lax
from jax.experimental import pallas as pl
from jax.experimental.pallas import tpu as pltpu
from jax.experimental.pallas import tpu_sc as plsc

N = 10000
D = 128
H = 128
R = 3
E = 160000

NC = 2
NS = 16
NW = NC * NS
EPT = E // NW
CH = 128
NCH = -(-EPT // CH)
EPT_PAD = NCH * CH
NPAD = 10112
ZROWS = NPAD // NS
NBUF = 2
NGRP = NCH // NBUF

_BN_SCALE = float(1.0 / (1.0 + 1e-5) ** 0.5)

BLK = 1000
GRID = N // BLK


def _make_sc_agg(with_counts):
  mesh = plsc.VectorSubcoreMesh(core_axis_name="c", subcore_axis_name="s")
  out_type = [jax.ShapeDtypeStruct((R, NC, NPAD, D), jnp.float32)]
  scratch = [
      pltpu.VMEM((NCH, CH), jnp.int32),
      pltpu.VMEM((NCH, CH), jnp.int32),
      pltpu.VMEM((NBUF, CH, D), jnp.float32),
      pltpu.VMEM_SHARED((NPAD, D), jnp.float32),
      pltpu.SemaphoreType.DMA((NBUF,)),
      pltpu.SemaphoreType.DMA((NBUF,)),
  ]
  if with_counts:
    out_type.append(jax.ShapeDtypeStruct((R, NC, NPAD, D), jnp.float32))

  @functools.partial(pl.kernel, mesh=mesh, out_type=tuple(out_type),
                     scratch_types=tuple(scratch))
  def sc_agg(*refs):
    if with_counts:
      (feat, src, dst, zfeat, ones, sums_out, cnt_out,
       src_v, dst_v, rows_v, acc_s, sem_g, sem_s) = refs
    else:
      (feat, src, dst, zfeat, sums_out,
       src_v, dst_v, rows_v, acc_s, sem_g, sem_s) = refs
    c = lax.axis_index("c")
    s = lax.axis_index("s")
    wid = c * NS + s
    my_rows = pl.ds(s * ZROWS, ZROWS)
    for r in range(R):
      pltpu.sync_copy(zfeat, acc_s.at[my_rows])
      pltpu.sync_copy(src.at[r, wid], src_v)
      pltpu.sync_copy(dst.at[r, wid], dst_v)
      for b in range(NBUF):
        pltpu.async_copy(feat.at[src_v.at[b]], rows_v.at[b], sem_g.at[b])
      plsc.subcore_barrier()

      def wait_gather(b):
        pltpu.make_async_copy(feat.at[src_v.at[0]], rows_v.at[b],
                              sem_g.at[b]).wait()

      def group(g, carry):
        descs = []
        for b in range(NBUF):
          j = g * NBUF + b
          wait_gather(b)
          descs.append(pltpu.async_copy(rows_v.at[b], acc_s.at[dst_v.at[j]],
                                        sem_s.at[b], add=True))
        for b in range(NBUF):
          descs[b].wait()
          pltpu.async_copy(feat.at[src_v.at[(g + 1) * NBUF + b]],
                           rows_v.at[b], sem_g.at[b])
        return carry

      lax.fori_loop(0, NGRP - 1, group, 0)
      descs = []
      for b in range(NBUF):
        wait_gather(b)
        descs.append(pltpu.async_copy(
            rows_v.at[b], acc_s.at[dst_v.at[(NGRP - 1) * NBUF + b]],
            sem_s.at[b], add=True))
      for d in descs:
        d.wait()
      plsc.subcore_barrier()
      pltpu.sync_copy(acc_s.at[my_rows], sums_out.at[r, c, my_rows])
      plsc.subcore_barrier()
    if with_counts:
      ones_v = rows_v.at[0]
      pltpu.sync_copy(ones, ones_v)
      for r in range(R):
        pltpu.sync_copy(zfeat, acc_s.at[my_rows])
        pltpu.sync_copy(dst.at[r, wid], dst_v)
        plsc.subcore_barrier()

        def cgroup(g, carry):
          descs = [
              pltpu.async_copy(ones_v, acc_s.at[dst_v.at[g * NBUF + b]],
                               sem_s.at[b], add=True)
              for b in range(NBUF)
          ]
          for d in descs:
            d.wait()
          return carry

        lax.fori_loop(0, NGRP, cgroup, 0)
        plsc.subcore_barrier()
        pltpu.sync_copy(acc_s.at[my_rows], cnt_out.at[r, c, my_rows])
        plsc.subcore_barrier()

  return sc_agg


_sc_agg_counts = _make_sc_agg(True)
_sc_agg_plain = _make_sc_agg(False)


def _leaky(v):
  return jnp.where(v >= 0, v, 0.01 * v)


def _hetero_acc(feat, sums_ref, cnt_ref, ws_ref, wn_ref, b_ref):
  acc = jnp.zeros(feat.shape, jnp.float32)
  for r in range(R):
    acc += jnp.dot(feat, ws_ref[r], preferred_element_type=jnp.float32)
    sm = sums_ref[r, 0] + sums_ref[r, 1]
    ct = cnt_ref[r, 0][:, :1] + cnt_ref[r, 1][:, :1]
    hn = sm * (1.0 / jnp.maximum(ct, 1.0))
    acc += jnp.dot(hn, wn_ref[r], preferred_element_type=jnp.float32)
    acc += b_ref[r]
  return acc


def _layer1_body(x_ref, sums_ref, cnt_ref, ws_ref, wn_ref, b_ref, g_ref,
                 bt_ref, out_ref):
  acc = _hetero_acc(x_ref[...], sums_ref, cnt_ref, ws_ref, wn_ref, b_ref)
  out_ref[...] = _leaky(g_ref[...] * _BN_SCALE * acc + bt_ref[...])


def _layer2_body(h1_ref, sums_ref, cnt_ref, ws_ref, wn_ref, b_ref, g_ref,
                 bt_ref, fw_ref, fb_ref, out_ref):
  h1 = h1_ref[...]
  acc = _hetero_acc(h1, sums_ref, cnt_ref, ws_ref, wn_ref, b_ref)
  h2 = _leaky(g_ref[...] * _BN_SCALE * acc + bt_ref[...] + h1)
  out_ref[...] = (jnp.dot(h1, fw_ref[0], preferred_element_type=jnp.float32)
                  + jnp.dot(h2, fw_ref[1], preferred_element_type=jnp.float32)
                  + fb_ref[...])


_COMMON_SPECS = [
    pl.BlockSpec((BLK, D), lambda i: (i, 0)),
    pl.BlockSpec((R, NC, BLK, D), lambda i: (0, 0, i, 0)),
    pl.BlockSpec((R, NC, BLK, D), lambda i: (0, 0, i, 0)),
    pl.BlockSpec((R, D, H), lambda i: (0, 0, 0)),
    pl.BlockSpec((R, D, H), lambda i: (0, 0, 0)),
    pl.BlockSpec((R, 1, H), lambda i: (0, 0, 0)),
    pl.BlockSpec((1, H), lambda i: (0, 0)),
    pl.BlockSpec((1, H), lambda i: (0, 0)),
]

_layer1_tc = pl.pallas_call(
    _layer1_body,
    grid=(GRID,),
    in_specs=_COMMON_SPECS,
    out_specs=pl.BlockSpec((BLK, H), lambda i: (i, 0)),
    out_shape=jax.ShapeDtypeStruct((N, H), jnp.float32),
)

_layer2_tc = pl.pallas_call(
    _layer2_body,
    grid=(GRID,),
    in_specs=_COMMON_SPECS + [
        pl.BlockSpec((2, H, H), lambda i: (0, 0, 0)),
        pl.BlockSpec((1, H), lambda i: (0, 0)),
    ],
    out_specs=pl.BlockSpec((BLK, H), lambda i: (i, 0)),
    out_shape=jax.ShapeDtypeStruct((N, H), jnp.float32),
)


def _prep_edges(ei):
  pad = EPT_PAD - EPT
  src = jnp.pad(ei[0].reshape(NW, EPT), ((0, 0), (0, pad)))
  dst = jnp.pad(ei[1].reshape(NW, EPT), ((0, 0), (0, pad)),
                constant_values=N)
  return src.reshape(NW, NCH, CH), dst.reshape(NW, NCH, CH)


def kernel(x, edge_index_0, edge_index_1, edge_index_2, W_self1, W_neigh1,
           b1, W_self2, W_neigh2, b2, bn1_gamma, bn1_beta, bn2_gamma,
           bn2_beta, fuse_W, fuse_b):
  pairs = [_prep_edges(e) for e in (edge_index_0, edge_index_1, edge_index_2)]
  src = jnp.stack([p[0] for p in pairs])
  dst = jnp.stack([p[1] for p in pairs])
  zfeat = jnp.zeros((ZROWS, D), jnp.float32)
  ones = jnp.ones((CH, D), jnp.float32)

  sums1, cnt = _sc_agg_counts(x, src, dst, zfeat, ones)
  h1 = _layer1_tc(x, sums1, cnt, W_self1, W_neigh1, b1.reshape(R, 1, H),
                  bn1_gamma.reshape(1, H), bn1_beta.reshape(1, H))
  (sums2,) = _sc_agg_plain(h1, src, dst, zfeat)
  out = _layer2_tc(h1, sums2, cnt, W_self2, W_neigh2, b2.reshape(R, 1, H),
                   bn2_gamma.reshape(1, H), bn2_beta.reshape(1, H),
                   fuse_W.reshape(2, H, H), fuse_b.reshape(1, H))
  return out


# speedup vs baseline: 2.8572x; 1.0192x over previous
import functools

import jax
import jax.numpy as jnp
from jax import lax
from jax.experimental import pallas as pl
from jax.experimental.pallas import tpu as pltpu
from jax.experimental.pallas import tpu_sc as plsc

N = 10000
D = 128
H = 128
R = 3
E = 160000

NC = 2
NS = 16
NW = NC * NS
EPT = E // NW
CH = 128
NCH = -(-EPT // CH)
EPT_PAD = NCH * CH
NPAD = 10112
ZROWS = NPAD // NS
NBUF = 2
SPLIT = 4
SCH = CH // SPLIT
NGRP = NCH // NBUF

_BN_SCALE = float(1.0 / (1.0 + 1e-5) ** 0.5)

BLK = 1000
GRID = N // BLK


def _make_sc_agg(with_counts):
  mesh = plsc.VectorSubcoreMesh(core_axis_name="c", subcore_axis_name="s")
  out_type = [jax.ShapeDtypeStruct((R, NC, NPAD, D), jnp.float32)]
  scratch = [
      pltpu.VMEM((NCH, CH), jnp.int32),
      pltpu.VMEM((NCH, CH), jnp.int32),
      pltpu.VMEM((NBUF, CH, D), jnp.float32),
      pltpu.VMEM_SHARED((NPAD, D), jnp.float32),
      pltpu.SemaphoreType.DMA((NBUF * SPLIT,)),
      pltpu.SemaphoreType.DMA((NBUF,)),
  ]
  if with_counts:
    out_type.append(jax.ShapeDtypeStruct((R, NC, NPAD, D), jnp.float32))

  @functools.partial(pl.kernel, mesh=mesh, out_type=tuple(out_type),
                     scratch_types=tuple(scratch))
  def sc_agg(*refs):
    if with_counts:
      (feat, src, dst, zfeat, ones, sums_out, cnt_out,
       src_v, dst_v, rows_v, acc_s, sem_g, sem_s) = refs
    else:
      (feat, src, dst, zfeat, sums_out,
       src_v, dst_v, rows_v, acc_s, sem_g, sem_s) = refs
    c = lax.axis_index("c")
    s = lax.axis_index("s")
    wid = c * NS + s
    my_rows = pl.ds(s * ZROWS, ZROWS)
    for r in range(R):
      pltpu.sync_copy(zfeat, acc_s.at[my_rows])
      pltpu.sync_copy(src.at[r, wid], src_v)
      pltpu.sync_copy(dst.at[r, wid], dst_v)
      def start_gather(b, j):
        for q in range(SPLIT):
          half = pl.ds(q * SCH, SCH)
          pltpu.async_copy(feat.at[src_v.at[j, half]],
                           rows_v.at[b, half], sem_g.at[b * SPLIT + q])

      def wait_gather(b):
        for q in range(SPLIT):
          half = pl.ds(q * SCH, SCH)
          pltpu.make_async_copy(feat.at[src_v.at[0, half]],
                                rows_v.at[b, half],
                                sem_g.at[b * SPLIT + q]).wait()

      for b in range(NBUF):
        start_gather(b, b)
      plsc.subcore_barrier()

      def group(g, carry):
        descs = []
        for b in range(NBUF):
          j = g * NBUF + b
          wait_gather(b)
          descs.append(pltpu.async_copy(rows_v.at[b], acc_s.at[dst_v.at[j]],
                                        sem_s.at[b], add=True))
        for b in range(NBUF):
          descs[b].wait()
          start_gather(b, (g + 1) * NBUF + b)
        return carry

      lax.fori_loop(0, NGRP - 1, group, 0)
      descs = []
      for b in range(NBUF):
        wait_gather(b)
        descs.append(pltpu.async_copy(
            rows_v.at[b], acc_s.at[dst_v.at[(NGRP - 1) * NBUF + b]],
            sem_s.at[b], add=True))
      for d in descs:
        d.wait()
      plsc.subcore_barrier()
      pltpu.sync_copy(acc_s.at[my_rows], sums_out.at[r, c, my_rows])
      plsc.subcore_barrier()
    if with_counts:
      ones_v = rows_v.at[0]
      pltpu.sync_copy(ones, ones_v)
      for r in range(R):
        pltpu.sync_copy(zfeat, acc_s.at[my_rows])
        pltpu.sync_copy(dst.at[r, wid], dst_v)
        plsc.subcore_barrier()

        def cgroup(g, carry):
          descs = [
              pltpu.async_copy(ones_v, acc_s.at[dst_v.at[g * NBUF + b]],
                               sem_s.at[b], add=True)
              for b in range(NBUF)
          ]
          for d in descs:
            d.wait()
          return carry

        lax.fori_loop(0, NGRP, cgroup, 0)
        plsc.subcore_barrier()
        pltpu.sync_copy(acc_s.at[my_rows], cnt_out.at[r, c, my_rows])
        plsc.subcore_barrier()

  return sc_agg


_sc_agg_counts = _make_sc_agg(True)
_sc_agg_plain = _make_sc_agg(False)


def _leaky(v):
  return jnp.where(v >= 0, v, 0.01 * v)


def _hetero_acc(feat, sums_ref, cnt_ref, ws_ref, wn_ref, b_ref):
  acc = jnp.zeros(feat.shape, jnp.float32)
  for r in range(R):
    acc += jnp.dot(feat, ws_ref[r], preferred_element_type=jnp.float32)
    sm = sums_ref[r, 0] + sums_ref[r, 1]
    ct = cnt_ref[r, 0][:, :1] + cnt_ref[r, 1][:, :1]
    hn = sm * (1.0 / jnp.maximum(ct, 1.0))
    acc += jnp.dot(hn, wn_ref[r], preferred_element_type=jnp.float32)
    acc += b_ref[r]
  return acc


def _layer1_body(x_ref, sums_ref, cnt_ref, ws_ref, wn_ref, b_ref, g_ref,
                 bt_ref, out_ref):
  acc = _hetero_acc(x_ref[...], sums_ref, cnt_ref, ws_ref, wn_ref, b_ref)
  out_ref[...] = _leaky(g_ref[...] * _BN_SCALE * acc + bt_ref[...])


def _layer2_body(h1_ref, sums_ref, cnt_ref, ws_ref, wn_ref, b_ref, g_ref,
                 bt_ref, fw_ref, fb_ref, out_ref):
  h1 = h1_ref[...]
  acc = _hetero_acc(h1, sums_ref, cnt_ref, ws_ref, wn_ref, b_ref)
  h2 = _leaky(g_ref[...] * _BN_SCALE * acc + bt_ref[...] + h1)
  out_ref[...] = (jnp.dot(h1, fw_ref[0], preferred_element_type=jnp.float32)
                  + jnp.dot(h2, fw_ref[1], preferred_element_type=jnp.float32)
                  + fb_ref[...])


_COMMON_SPECS = [
    pl.BlockSpec((BLK, D), lambda i: (i, 0)),
    pl.BlockSpec((R, NC, BLK, D), lambda i: (0, 0, i, 0)),
    pl.BlockSpec((R, NC, BLK, D), lambda i: (0, 0, i, 0)),
    pl.BlockSpec((R, D, H), lambda i: (0, 0, 0)),
    pl.BlockSpec((R, D, H), lambda i: (0, 0, 0)),
    pl.BlockSpec((R, 1, H), lambda i: (0, 0, 0)),
    pl.BlockSpec((1, H), lambda i: (0, 0)),
    pl.BlockSpec((1, H), lambda i: (0, 0)),
]

_layer1_tc = pl.pallas_call(
    _layer1_body,
    grid=(GRID,),
    in_specs=_COMMON_SPECS,
    out_specs=pl.BlockSpec((BLK, H), lambda i: (i, 0)),
    out_shape=jax.ShapeDtypeStruct((N, H), jnp.float32),
)

_layer2_tc = pl.pallas_call(
    _layer2_body,
    grid=(GRID,),
    in_specs=_COMMON_SPECS + [
        pl.BlockSpec((2, H, H), lambda i: (0, 0, 0)),
        pl.BlockSpec((1, H), lambda i: (0, 0)),
    ],
    out_specs=pl.BlockSpec((BLK, H), lambda i: (i, 0)),
    out_shape=jax.ShapeDtypeStruct((N, H), jnp.float32),
)


def _prep_edges(ei):
  pad = EPT_PAD - EPT
  src = jnp.pad(ei[0].reshape(NW, EPT), ((0, 0), (0, pad)))
  dst = jnp.pad(ei[1].reshape(NW, EPT), ((0, 0), (0, pad)),
                constant_values=N)
  return src.reshape(NW, NCH, CH), dst.reshape(NW, NCH, CH)


def kernel(x, edge_index_0, edge_index_1, edge_index_2, W_self1, W_neigh1,
           b1, W_self2, W_neigh2, b2, bn1_gamma, bn1_beta, bn2_gamma,
           bn2_beta, fuse_W, fuse_b):
  pairs = [_prep_edges(e) for e in (edge_index_0, edge_index_1, edge_index_2)]
  src = jnp.stack([p[0] for p in pairs])
  dst = jnp.stack([p[1] for p in pairs])
  zfeat = jnp.zeros((ZROWS, D), jnp.float32)
  ones = jnp.ones((CH, D), jnp.float32)

  sums1, cnt = _sc_agg_counts(x, src, dst, zfeat, ones)
  h1 = _layer1_tc(x, sums1, cnt, W_self1, W_neigh1, b1.reshape(R, 1, H),
                  bn1_gamma.reshape(1, H), bn1_beta.reshape(1, H))
  (sums2,) = _sc_agg_plain(h1, src, dst, zfeat)
  out = _layer2_tc(h1, sums2, cnt, W_self2, W_neigh2, b2.reshape(R, 1, H),
                   bn2_gamma.reshape(1, H), bn2_beta.reshape(1, H),
                   fuse_W.reshape(2, H, H), fuse_b.reshape(1, H))
  return out


# speedup vs baseline: 2.9163x; 1.0207x over previous
import functools

import jax
import jax.numpy as jnp
from jax import lax
from jax.experimental import pallas as pl
from jax.experimental.pallas import tpu as pltpu
from jax.experimental.pallas import tpu_sc as plsc

N = 10000
D = 128
H = 128
R = 3
E = 160000

NC = 2
NS = 16
NW = NC * NS
EPT = E // NW
CH = 128
NCH = -(-EPT // CH)
EPT_PAD = NCH * CH
NPAD = 10112
ZROWS = NPAD // NS
NBUF = 2
SPLIT = 4
SCH = CH // SPLIT
NGRP = NCH // NBUF

_BN_SCALE = float(1.0 / (1.0 + 1e-5) ** 0.5)

BLK = 1000
GRID = N // BLK


def _make_sc_agg(with_counts):
  mesh = plsc.VectorSubcoreMesh(core_axis_name="c", subcore_axis_name="s")
  out_type = [jax.ShapeDtypeStruct((R, NC, NPAD, D), jnp.float32)]
  scratch = [
      pltpu.VMEM((NCH, CH), jnp.int32),
      pltpu.VMEM((NCH, CH), jnp.int32),
      pltpu.VMEM((NBUF, CH, D), jnp.float32),
      pltpu.VMEM_SHARED((NPAD, D), jnp.float32),
      pltpu.SemaphoreType.DMA((NBUF * SPLIT,)),
      pltpu.SemaphoreType.DMA((NBUF,)),
      pltpu.SemaphoreType.DMA,
  ]
  if with_counts:
    out_type.append(jax.ShapeDtypeStruct((R, NC, NPAD, D), jnp.float32))

  @functools.partial(pl.kernel, mesh=mesh, out_type=tuple(out_type),
                     scratch_types=tuple(scratch))
  def sc_agg(*refs):
    if with_counts:
      (feat, src, dst, zfeat, ones, sums_out, cnt_out,
       src_v, dst_v, rows_v, acc_s, sem_g, sem_s, sem_w) = refs
    else:
      (feat, src, dst, zfeat, sums_out,
       src_v, dst_v, rows_v, acc_s, sem_g, sem_s, sem_w) = refs
    c = lax.axis_index("c")
    s = lax.axis_index("s")
    wid = c * NS + s
    my_rows = pl.ds(s * ZROWS, ZROWS)

    def start_gather(b, j):
      for q in range(SPLIT):
        half = pl.ds(q * SCH, SCH)
        pltpu.async_copy(feat.at[src_v.at[j, half]],
                         rows_v.at[b, half], sem_g.at[b * SPLIT + q])

    def wait_gather(b):
      for q in range(SPLIT):
        half = pl.ds(q * SCH, SCH)
        pltpu.make_async_copy(feat.at[src_v.at[0, half]],
                              rows_v.at[b, half],
                              sem_g.at[b * SPLIT + q]).wait()

    def load_idx(r, also_src):
      if also_src:
        pltpu.sync_copy(src.at[r, wid], src_v)
      pltpu.sync_copy(dst.at[r, wid], dst_v)

    pltpu.sync_copy(zfeat, acc_s.at[my_rows])
    load_idx(0, True)
    for b in range(NBUF):
      start_gather(b, b)
    plsc.subcore_barrier()

    def transition(out_slice, next_fill):
      wo = pltpu.async_copy(acc_s.at[my_rows], out_slice, sem_w)
      if next_fill is not None:
        next_fill()
      wo.wait()
      if next_fill is not None:
        pltpu.async_copy(zfeat, acc_s.at[my_rows], sem_w).wait()
      plsc.subcore_barrier()

    for r in range(R):
      def group(g, carry):
        descs = []
        for b in range(NBUF):
          j = g * NBUF + b
          wait_gather(b)
          descs.append(pltpu.async_copy(rows_v.at[b], acc_s.at[dst_v.at[j]],
                                        sem_s.at[b], add=True))
        for b in range(NBUF):
          descs[b].wait()
          start_gather(b, (g + 1) * NBUF + b)
        return carry

      lax.fori_loop(0, NGRP - 1, group, 0)
      descs = []
      for b in range(NBUF):
        wait_gather(b)
        descs.append(pltpu.async_copy(
            rows_v.at[b], acc_s.at[dst_v.at[(NGRP - 1) * NBUF + b]],
            sem_s.at[b], add=True))
      for d in descs:
        d.wait()
      plsc.subcore_barrier()

      if r < R - 1:
        def next_fill(r=r):
          load_idx(r + 1, True)
          for b in range(NBUF):
            start_gather(b, b)
      elif with_counts:
        def next_fill():
          pltpu.sync_copy(ones, rows_v.at[0])
          load_idx(0, False)
      else:
        next_fill = None
      transition(sums_out.at[r, c, my_rows], next_fill)

    if with_counts:
      ones_v = rows_v.at[0]
      for r in range(R):
        def cgroup(g, carry):
          descs = [
              pltpu.async_copy(ones_v, acc_s.at[dst_v.at[g * NBUF + b]],
                               sem_s.at[b], add=True)
              for b in range(NBUF)
          ]
          for d in descs:
            d.wait()
          return carry

        lax.fori_loop(0, NGRP, cgroup, 0)
        plsc.subcore_barrier()

        if r < R - 1:
          def next_fill(r=r):
            load_idx(r + 1, False)
        else:
          next_fill = None
        transition(cnt_out.at[r, c, my_rows], next_fill)

  return sc_agg


_sc_agg_counts = _make_sc_agg(True)
_sc_agg_plain = _make_sc_agg(False)


def _leaky(v):
  return jnp.where(v >= 0, v, 0.01 * v)


def _hetero_acc(feat, sums_ref, cnt_ref, ws_ref, wn_ref, b_ref):
  acc = jnp.zeros(feat.shape, jnp.float32)
  for r in range(R):
    acc += jnp.dot(feat, ws_ref[r], preferred_element_type=jnp.float32)
    sm = sums_ref[r, 0] + sums_ref[r, 1]
    ct = cnt_ref[r, 0][:, :1] + cnt_ref[r, 1][:, :1]
    hn = sm * (1.0 / jnp.maximum(ct, 1.0))
    acc += jnp.dot(hn, wn_ref[r], preferred_element_type=jnp.float32)
    acc += b_ref[r]
  return acc


def _layer1_body(x_ref, sums_ref, cnt_ref, ws_ref, wn_ref, b_ref, g_ref,
                 bt_ref, out_ref):
  acc = _hetero_acc(x_ref[...], sums_ref, cnt_ref, ws_ref, wn_ref, b_ref)
  out_ref[...] = _leaky(g_ref[...] * _BN_SCALE * acc + bt_ref[...])


def _layer2_body(h1_ref, sums_ref, cnt_ref, ws_ref, wn_ref, b_ref, g_ref,
                 bt_ref, fw_ref, fb_ref, out_ref):
  h1 = h1_ref[...]
  acc = _hetero_acc(h1, sums_ref, cnt_ref, ws_ref, wn_ref, b_ref)
  h2 = _leaky(g_ref[...] * _BN_SCALE * acc + bt_ref[...] + h1)
  out_ref[...] = (jnp.dot(h1, fw_ref[0], preferred_element_type=jnp.float32)
                  + jnp.dot(h2, fw_ref[1], preferred_element_type=jnp.float32)
                  + fb_ref[...])


_COMMON_SPECS = [
    pl.BlockSpec((BLK, D), lambda i: (i, 0)),
    pl.BlockSpec((R, NC, BLK, D), lambda i: (0, 0, i, 0)),
    pl.BlockSpec((R, NC, BLK, D), lambda i: (0, 0, i, 0)),
    pl.BlockSpec((R, D, H), lambda i: (0, 0, 0)),
    pl.BlockSpec((R, D, H), lambda i: (0, 0, 0)),
    pl.BlockSpec((R, 1, H), lambda i: (0, 0, 0)),
    pl.BlockSpec((1, H), lambda i: (0, 0)),
    pl.BlockSpec((1, H), lambda i: (0, 0)),
]

_layer1_tc = pl.pallas_call(
    _layer1_body,
    grid=(GRID,),
    in_specs=_COMMON_SPECS,
    out_specs=pl.BlockSpec((BLK, H), lambda i: (i, 0)),
    out_shape=jax.ShapeDtypeStruct((N, H), jnp.float32),
)

_layer2_tc = pl.pallas_call(
    _layer2_body,
    grid=(GRID,),
    in_specs=_COMMON_SPECS + [
        pl.BlockSpec((2, H, H), lambda i: (0, 0, 0)),
        pl.BlockSpec((1, H), lambda i: (0, 0)),
    ],
    out_specs=pl.BlockSpec((BLK, H), lambda i: (i, 0)),
    out_shape=jax.ShapeDtypeStruct((N, H), jnp.float32),
)


def _prep_edges(ei):
  pad = EPT_PAD - EPT
  src = jnp.pad(ei[0].reshape(NW, EPT), ((0, 0), (0, pad)))
  dst = jnp.pad(ei[1].reshape(NW, EPT), ((0, 0), (0, pad)),
                constant_values=N)
  return src.reshape(NW, NCH, CH), dst.reshape(NW, NCH, CH)


def kernel(x, edge_index_0, edge_index_1, edge_index_2, W_self1, W_neigh1,
           b1, W_self2, W_neigh2, b2, bn1_gamma, bn1_beta, bn2_gamma,
           bn2_beta, fuse_W, fuse_b):
  pairs = [_prep_edges(e) for e in (edge_index_0, edge_index_1, edge_index_2)]
  src = jnp.stack([p[0] for p in pairs])
  dst = jnp.stack([p[1] for p in pairs])
  zfeat = jnp.zeros((ZROWS, D), jnp.float32)
  ones = jnp.ones((CH, D), jnp.float32)

  sums1, cnt = _sc_agg_counts(x, src, dst, zfeat, ones)
  h1 = _layer1_tc(x, sums1, cnt, W_self1, W_neigh1, b1.reshape(R, 1, H),
                  bn1_gamma.reshape(1, H), bn1_beta.reshape(1, H))
  (sums2,) = _sc_agg_plain(h1, src, dst, zfeat)
  out = _layer2_tc(h1, sums2, cnt, W_self2, W_neigh2, b2.reshape(R, 1, H),
                   bn2_gamma.reshape(1, H), bn2_beta.reshape(1, H),
                   fuse_W.reshape(2, H, H), fuse_b.reshape(1, H))
  return out


# speedup vs baseline: 2.9541x; 1.0129x over previous
import functools

import jax
import jax.numpy as jnp
from jax import lax
from jax.experimental import pallas as pl
from jax.experimental.pallas import tpu as pltpu
from jax.experimental.pallas import tpu_sc as plsc

N = 10000
D = 128
H = 128
R = 3
E = 160000

NC = 2
NS = 16
NW = NC * NS
EPT = E // NW
CH = 128
NCH = -(-EPT // CH)
EPT_PAD = NCH * CH
NPAD = 10112
ZROWS = NPAD // NS
NBUF = 2
SPLIT = 8
SCH = CH // SPLIT
NGRP = NCH // NBUF

_BN_SCALE = float(1.0 / (1.0 + 1e-5) ** 0.5)

BLK = 1000
GRID = N // BLK


def _make_sc_agg(with_counts):
  mesh = plsc.VectorSubcoreMesh(core_axis_name="c", subcore_axis_name="s")
  out_type = [jax.ShapeDtypeStruct((R, NC, NPAD, D), jnp.float32)]
  scratch = [
      pltpu.VMEM((NCH, CH), jnp.int32),
      pltpu.VMEM((NCH, CH), jnp.int32),
      pltpu.VMEM((NBUF, CH, D), jnp.float32),
      pltpu.VMEM_SHARED((NPAD, D), jnp.float32),
      pltpu.SemaphoreType.DMA((NBUF * SPLIT,)),
      pltpu.SemaphoreType.DMA((NBUF,)),
      pltpu.SemaphoreType.DMA,
  ]
  if with_counts:
    out_type.append(jax.ShapeDtypeStruct((R, NC, NPAD, D), jnp.float32))

  @functools.partial(pl.kernel, mesh=mesh, out_type=tuple(out_type),
                     scratch_types=tuple(scratch))
  def sc_agg(*refs):
    if with_counts:
      (feat, src, dst, zfeat, ones, sums_out, cnt_out,
       src_v, dst_v, rows_v, acc_s, sem_g, sem_s, sem_w) = refs
    else:
      (feat, src, dst, zfeat, sums_out,
       src_v, dst_v, rows_v, acc_s, sem_g, sem_s, sem_w) = refs
    c = lax.axis_index("c")
    s = lax.axis_index("s")
    wid = c * NS + s
    my_rows = pl.ds(s * ZROWS, ZROWS)

    def start_gather(b, j):
      for q in range(SPLIT):
        half = pl.ds(q * SCH, SCH)
        pltpu.async_copy(feat.at[src_v.at[j, half]],
                         rows_v.at[b, half], sem_g.at[b * SPLIT + q])

    def wait_gather(b):
      for q in range(SPLIT):
        half = pl.ds(q * SCH, SCH)
        pltpu.make_async_copy(feat.at[src_v.at[0, half]],
                              rows_v.at[b, half],
                              sem_g.at[b * SPLIT + q]).wait()

    def load_idx(r, also_src):
      if also_src:
        pltpu.sync_copy(src.at[r, wid], src_v)
      pltpu.sync_copy(dst.at[r, wid], dst_v)

    pltpu.sync_copy(zfeat, acc_s.at[my_rows])
    load_idx(0, True)
    for b in range(NBUF):
      start_gather(b, b)
    plsc.subcore_barrier()

    def transition(out_slice, next_fill):
      wo = pltpu.async_copy(acc_s.at[my_rows], out_slice, sem_w)
      if next_fill is not None:
        next_fill()
      wo.wait()
      if next_fill is not None:
        pltpu.async_copy(zfeat, acc_s.at[my_rows], sem_w).wait()
      plsc.subcore_barrier()

    for r in range(R):
      def group(g, carry):
        descs = []
        for b in range(NBUF):
          j = g * NBUF + b
          wait_gather(b)
          descs.append(pltpu.async_copy(rows_v.at[b], acc_s.at[dst_v.at[j]],
                                        sem_s.at[b], add=True))
        for b in range(NBUF):
          descs[b].wait()
          start_gather(b, (g + 1) * NBUF + b)
        return carry

      lax.fori_loop(0, NGRP - 1, group, 0)
      descs = []
      for b in range(NBUF):
        wait_gather(b)
        descs.append(pltpu.async_copy(
            rows_v.at[b], acc_s.at[dst_v.at[(NGRP - 1) * NBUF + b]],
            sem_s.at[b], add=True))
      for d in descs:
        d.wait()
      plsc.subcore_barrier()

      if r < R - 1:
        def next_fill(r=r):
          load_idx(r + 1, True)
          for b in range(NBUF):
            start_gather(b, b)
      elif with_counts:
        def next_fill():
          pltpu.sync_copy(ones, rows_v.at[0])
          load_idx(0, False)
      else:
        next_fill = None
      transition(sums_out.at[r, c, my_rows], next_fill)

    if with_counts:
      ones_v = rows_v.at[0]
      for r in range(R):
        def cgroup(g, carry):
          descs = [
              pltpu.async_copy(ones_v, acc_s.at[dst_v.at[g * NBUF + b]],
                               sem_s.at[b], add=True)
              for b in range(NBUF)
          ]
          for d in descs:
            d.wait()
          return carry

        lax.fori_loop(0, NGRP, cgroup, 0)
        plsc.subcore_barrier()

        if r < R - 1:
          def next_fill(r=r):
            load_idx(r + 1, False)
        else:
          next_fill = None
        transition(cnt_out.at[r, c, my_rows], next_fill)

  return sc_agg


_sc_agg_counts = _make_sc_agg(True)
_sc_agg_plain = _make_sc_agg(False)


def _leaky(v):
  return jnp.where(v >= 0, v, 0.01 * v)


def _hetero_acc(feat, sums_ref, cnt_ref, ws_ref, wn_ref, b_ref):
  acc = jnp.zeros(feat.shape, jnp.float32)
  for r in range(R):
    acc += jnp.dot(feat, ws_ref[r], preferred_element_type=jnp.float32)
    sm = sums_ref[r, 0] + sums_ref[r, 1]
    ct = cnt_ref[r, 0][:, :1] + cnt_ref[r, 1][:, :1]
    hn = sm * (1.0 / jnp.maximum(ct, 1.0))
    acc += jnp.dot(hn, wn_ref[r], preferred_element_type=jnp.float32)
    acc += b_ref[r]
  return acc


def _layer1_body(x_ref, sums_ref, cnt_ref, ws_ref, wn_ref, b_ref, g_ref,
                 bt_ref, out_ref):
  acc = _hetero_acc(x_ref[...], sums_ref, cnt_ref, ws_ref, wn_ref, b_ref)
  out_ref[...] = _leaky(g_ref[...] * _BN_SCALE * acc + bt_ref[...])


def _layer2_body(h1_ref, sums_ref, cnt_ref, ws_ref, wn_ref, b_ref, g_ref,
                 bt_ref, fw_ref, fb_ref, out_ref):
  h1 = h1_ref[...]
  acc = _hetero_acc(h1, sums_ref, cnt_ref, ws_ref, wn_ref, b_ref)
  h2 = _leaky(g_ref[...] * _BN_SCALE * acc + bt_ref[...] + h1)
  out_ref[...] = (jnp.dot(h1, fw_ref[0], preferred_element_type=jnp.float32)
                  + jnp.dot(h2, fw_ref[1], preferred_element_type=jnp.float32)
                  + fb_ref[...])


_COMMON_SPECS = [
    pl.BlockSpec((BLK, D), lambda i: (i, 0)),
    pl.BlockSpec((R, NC, BLK, D), lambda i: (0, 0, i, 0)),
    pl.BlockSpec((R, NC, BLK, D), lambda i: (0, 0, i, 0)),
    pl.BlockSpec((R, D, H), lambda i: (0, 0, 0)),
    pl.BlockSpec((R, D, H), lambda i: (0, 0, 0)),
    pl.BlockSpec((R, 1, H), lambda i: (0, 0, 0)),
    pl.BlockSpec((1, H), lambda i: (0, 0)),
    pl.BlockSpec((1, H), lambda i: (0, 0)),
]

_layer1_tc = pl.pallas_call(
    _layer1_body,
    grid=(GRID,),
    in_specs=_COMMON_SPECS,
    out_specs=pl.BlockSpec((BLK, H), lambda i: (i, 0)),
    out_shape=jax.ShapeDtypeStruct((N, H), jnp.float32),
)

_layer2_tc = pl.pallas_call(
    _layer2_body,
    grid=(GRID,),
    in_specs=_COMMON_SPECS + [
        pl.BlockSpec((2, H, H), lambda i: (0, 0, 0)),
        pl.BlockSpec((1, H), lambda i: (0, 0)),
    ],
    out_specs=pl.BlockSpec((BLK, H), lambda i: (i, 0)),
    out_shape=jax.ShapeDtypeStruct((N, H), jnp.float32),
)


def _prep_edges(ei):
  pad = EPT_PAD - EPT
  src = jnp.pad(ei[0].reshape(NW, EPT), ((0, 0), (0, pad)))
  dst = jnp.pad(ei[1].reshape(NW, EPT), ((0, 0), (0, pad)),
                constant_values=N)
  return src.reshape(NW, NCH, CH), dst.reshape(NW, NCH, CH)


def kernel(x, edge_index_0, edge_index_1, edge_index_2, W_self1, W_neigh1,
           b1, W_self2, W_neigh2, b2, bn1_gamma, bn1_beta, bn2_gamma,
           bn2_beta, fuse_W, fuse_b):
  pairs = [_prep_edges(e) for e in (edge_index_0, edge_index_1, edge_index_2)]
  src = jnp.stack([p[0] for p in pairs])
  dst = jnp.stack([p[1] for p in pairs])
  zfeat = jnp.zeros((ZROWS, D), jnp.float32)
  ones = jnp.ones((CH, D), jnp.float32)

  sums1, cnt = _sc_agg_counts(x, src, dst, zfeat, ones)
  h1 = _layer1_tc(x, sums1, cnt, W_self1, W_neigh1, b1.reshape(R, 1, H),
                  bn1_gamma.reshape(1, H), bn1_beta.reshape(1, H))
  (sums2,) = _sc_agg_plain(h1, src, dst, zfeat)
  out = _layer2_tc(h1, sums2, cnt, W_self2, W_neigh2, b2.reshape(R, 1, H),
                   bn2_gamma.reshape(1, H), bn2_beta.reshape(1, H),
                   fuse_W.reshape(2, H, H), fuse_b.reshape(1, H))
  return out
